# Initial kernel scaffold; baseline (speedup 1.0000x reference)
import jax
import jax.numpy as jnp
from jax.experimental import pallas as pl


def kernel(x, wi_f, wh_f, b_f, wi_b, wh_b, b_b, wl_f, wl_b, b_lin):
    raise NotImplementedError("write your pallas kernel here")


# speedup vs baseline: 1.7393x; 1.7393x over previous
import functools

import jax
import jax.numpy as jnp
from jax.experimental import pallas as pl
from jax.experimental.pallas import tpu as pltpu


def _bilstm_step_body(T, B, I, H, O,
                      x_ref,
                      wi_ref,
                      wh_ref,
                      b_ref,
                      wl_ref,
                      o_ref,
                      h_ref,
                      c_ref):
    s = pl.program_id(1)

    @pl.when(s == 0)
    def _init():
        h_ref[...] = jnp.zeros((B, H), jnp.float32)
        c_ref[...] = jnp.zeros((B, H), jnp.float32)

    xs = x_ref[...].astype(jnp.bfloat16)
    g_in = jnp.dot(xs, wi_ref[0], preferred_element_type=jnp.float32) + b_ref[0]

    h = h_ref[...]
    c = c_ref[...]
    gates = g_in + jnp.dot(h, wh_ref[0], preferred_element_type=jnp.float32)

    col = jax.lax.broadcasted_iota(jnp.int32, (1, 4 * H), 1)
    scale = jnp.where((col >= 2 * H) & (col < 3 * H), 1.0, 0.5).astype(jnp.float32)
    th = jnp.tanh(gates * scale)
    i_g = th[:, 0 * H:1 * H] * 0.5 + 0.5
    f_g = th[:, 1 * H:2 * H] * 0.5 + 0.5
    g_g = th[:, 2 * H:3 * H]
    o_g = th[:, 3 * H:4 * H] * 0.5 + 0.5

    c2 = f_g * c + i_g * g_g
    h2 = o_g * jnp.tanh(c2)
    h_ref[...] = h2
    c_ref[...] = c2

    o_ref[0, 0] = jnp.dot(h2, wl_ref[0], preferred_element_type=jnp.float32)


@jax.jit
def kernel(x, wi_f, wh_f, b_f, wi_b, wh_b, b_b, wl_f, wl_b, b_lin):
    B, T, I = x.shape
    H = wh_f.shape[0]
    O = b_lin.shape[-1]
    f32 = jnp.float32

    wi = jnp.stack([wi_f, wi_b]).astype(jnp.bfloat16)
    wh = jnp.stack([wh_f, wh_b]).astype(f32)
    b = jnp.stack([b_f, b_b]).astype(f32)
    wl = jnp.stack([wl_f, wl_b]).astype(f32)

    x2 = x.reshape(B, T * I)

    def t_of(d, s):
        return jnp.where(d == 0, s, T - 1 - s)

    parts = pl.pallas_call(
        functools.partial(_bilstm_step_body, T, B, I, H, O),
        out_shape=jax.ShapeDtypeStruct((2, T, B, O), f32),
        grid_spec=pltpu.PrefetchScalarGridSpec(
            num_scalar_prefetch=0,
            grid=(2, T),
            in_specs=[
                pl.BlockSpec((B, I), lambda d, s: (0, t_of(d, s))),
                pl.BlockSpec((1, I, 4 * H), lambda d, s: (d, 0, 0)),
                pl.BlockSpec((1, H, 4 * H), lambda d, s: (d, 0, 0)),
                pl.BlockSpec((1, 1, 4 * H), lambda d, s: (d, 0, 0)),
                pl.BlockSpec((1, H, O), lambda d, s: (d, 0, 0)),
            ],
            out_specs=pl.BlockSpec((1, 1, B, O),
                                   lambda d, s: (d, t_of(d, s), 0, 0)),
            scratch_shapes=[pltpu.VMEM((B, H), f32), pltpu.VMEM((B, H), f32)],
        ),
        compiler_params=pltpu.CompilerParams(
            dimension_semantics=("parallel", "arbitrary")),
    )(x2, wi, wh, b, wl)

    out_tm = parts[0] + parts[1] + b_lin
    return jnp.transpose(out_tm, (1, 0, 2))


# speedup vs baseline: 1.7482x; 1.0051x over previous
import functools

import jax
import jax.numpy as jnp
from jax.experimental import pallas as pl
from jax.experimental.pallas import tpu as pltpu


def _bilstm_step_body(T, B, I, H, O,
                      x_ref,
                      wi_f_ref,
                      wi_b_ref,
                      wh_f_ref,
                      wh_b_ref,
                      b_f_ref,
                      b_b_ref,
                      wl_f_ref,
                      wl_b_ref,
                      o_ref,
                      wi_scr,
                      wh_scr,
                      b_scr,
                      wl_scr,
                      h_ref,
                      c_ref):
    d = pl.program_id(0)
    s = pl.program_id(1)

    @pl.when(s == 0)
    def _init():
        fwd = d == 0
        wi_scr[...] = jnp.where(fwd, wi_f_ref[...], wi_b_ref[...]).astype(jnp.bfloat16)
        wh_scr[...] = jnp.where(fwd, wh_f_ref[...], wh_b_ref[...])
        b_scr[...] = jnp.where(fwd, b_f_ref[...], b_b_ref[...])
        wl_scr[...] = jnp.where(fwd, wl_f_ref[...], wl_b_ref[...])
        h_ref[...] = jnp.zeros((B, H), jnp.float32)
        c_ref[...] = jnp.zeros((B, H), jnp.float32)

    xs = x_ref[...].astype(jnp.bfloat16)
    g_in = jnp.dot(xs, wi_scr[...], preferred_element_type=jnp.float32) + b_scr[...]

    h = h_ref[...]
    c = c_ref[...]
    gates = g_in + jnp.dot(h, wh_scr[...], preferred_element_type=jnp.float32)

    col = jax.lax.broadcasted_iota(jnp.int32, (1, 4 * H), 1)
    scale = jnp.where((col >= 2 * H) & (col < 3 * H), 1.0, 0.5).astype(jnp.float32)
    th = jnp.tanh(gates * scale)
    i_g = th[:, 0 * H:1 * H] * 0.5 + 0.5
    f_g = th[:, 1 * H:2 * H] * 0.5 + 0.5
    g_g = th[:, 2 * H:3 * H]
    o_g = th[:, 3 * H:4 * H] * 0.5 + 0.5

    c2 = f_g * c + i_g * g_g
    h2 = o_g * jnp.tanh(c2)
    h_ref[...] = h2
    c_ref[...] = c2

    o_ref[0, 0] = jnp.dot(h2, wl_scr[...], preferred_element_type=jnp.float32)


@jax.jit
def kernel(x, wi_f, wh_f, b_f, wi_b, wh_b, b_b, wl_f, wl_b, b_lin):
    B, T, I = x.shape
    H = wh_f.shape[0]
    O = b_lin.shape[-1]
    f32 = jnp.float32

    x2 = x.reshape(B, T * I)

    def t_of(d, s):
        return jnp.where(d == 0, s, T - 1 - s)

    def whole(shape):
        return pl.BlockSpec(shape, lambda d, s, _n=len(shape): (0,) * _n)

    parts = pl.pallas_call(
        functools.partial(_bilstm_step_body, T, B, I, H, O),
        out_shape=jax.ShapeDtypeStruct((2, T, B, O), f32),
        grid_spec=pltpu.PrefetchScalarGridSpec(
            num_scalar_prefetch=0,
            grid=(2, T),
            in_specs=[
                pl.BlockSpec((B, I), lambda d, s: (0, t_of(d, s))),
                whole((I, 4 * H)),
                whole((I, 4 * H)),
                whole((H, 4 * H)),
                whole((H, 4 * H)),
                whole((1, 4 * H)),
                whole((1, 4 * H)),
                whole((H, O)),
                whole((H, O)),
            ],
            out_specs=pl.BlockSpec((1, 1, B, O),
                                   lambda d, s: (d, t_of(d, s), 0, 0)),
            scratch_shapes=[
                pltpu.VMEM((I, 4 * H), jnp.bfloat16),
                pltpu.VMEM((H, 4 * H), f32),
                pltpu.VMEM((1, 4 * H), f32),
                pltpu.VMEM((H, O), f32),
                pltpu.VMEM((B, H), f32),
                pltpu.VMEM((B, H), f32),
            ],
        ),
        compiler_params=pltpu.CompilerParams(
            dimension_semantics=("parallel", "arbitrary")),
    )(x2, wi_f, wi_b, wh_f, wh_b, b_f, b_b, wl_f, wl_b)

    out_tm = parts[0] + parts[1] + b_lin
    return jnp.transpose(out_tm, (1, 0, 2))


# speedup vs baseline: 1.7626x; 1.0082x over previous
import functools

import jax
import jax.numpy as jnp
from jax.experimental import pallas as pl
from jax.experimental.pallas import tpu as pltpu


def _bilstm_step_body(T, B, I, H, O,
                      x_ref,
                      wi_f_ref,
                      wi_b_ref,
                      wh_f_ref,
                      wh_b_ref,
                      b_f_ref,
                      b_b_ref,
                      wl_f_ref,
                      wl_b_ref,
                      o_ref,
                      wi_scr,
                      wh_scr,
                      b_scr,
                      wl_scr,
                      h_ref,
                      c_ref):
    d = pl.program_id(0)
    s = pl.program_id(1)

    @pl.when(s == 0)
    def _init():
        fwd = d == 0
        wi_scr[...] = jnp.where(fwd, wi_f_ref[...], wi_b_ref[...]).astype(jnp.bfloat16)
        wh_scr[...] = jnp.where(fwd, wh_f_ref[...], wh_b_ref[...])
        b_scr[...] = jnp.where(fwd, b_f_ref[...], b_b_ref[...])
        wl_scr[...] = jnp.where(fwd, wl_f_ref[...], wl_b_ref[...])
        h_ref[...] = jnp.zeros((B, H), jnp.float32)
        c_ref[...] = jnp.zeros((B, H), jnp.float32)

    xs = x_ref[...].astype(jnp.bfloat16)
    g_in = jnp.dot(xs, wi_scr[...], preferred_element_type=jnp.float32) + b_scr[...]

    h = h_ref[...]
    c = c_ref[...]
    gates = g_in + jnp.dot(h, wh_scr[...], preferred_element_type=jnp.float32)

    col = jax.lax.broadcasted_iota(jnp.int32, (1, 4 * H), 1)
    scale = jnp.where((col >= 2 * H) & (col < 3 * H), 1.0, 0.5).astype(jnp.float32)
    th = jnp.tanh(gates * scale)
    i_g = th[:, 0 * H:1 * H] * 0.5 + 0.5
    f_g = th[:, 1 * H:2 * H] * 0.5 + 0.5
    g_g = th[:, 2 * H:3 * H]
    o_g = th[:, 3 * H:4 * H] * 0.5 + 0.5

    c2 = f_g * c + i_g * g_g
    h2 = o_g * jnp.tanh(c2)
    h_ref[...] = h2
    c_ref[...] = c2

    o_ref[0] = jnp.dot(h2, wl_scr[...], preferred_element_type=jnp.float32)


@jax.jit
def kernel(x, wi_f, wh_f, b_f, wi_b, wh_b, b_b, wl_f, wl_b, b_lin):
    B, T, I = x.shape
    H = wh_f.shape[0]
    O = b_lin.shape[-1]
    f32 = jnp.float32

    x2 = x.reshape(B, T * I)

    def t_of(d, s):
        return jnp.where(d == 0, s, T - 1 - s)

    def whole(shape):
        return pl.BlockSpec(shape, lambda d, s, _n=len(shape): (0,) * _n)

    parts = pl.pallas_call(
        functools.partial(_bilstm_step_body, T, B, I, H, O),
        out_shape=jax.ShapeDtypeStruct((2, B, T * O), f32),
        grid_spec=pltpu.PrefetchScalarGridSpec(
            num_scalar_prefetch=0,
            grid=(2, T),
            in_specs=[
                pl.BlockSpec((B, I), lambda d, s: (0, t_of(d, s))),
                whole((I, 4 * H)),
                whole((I, 4 * H)),
                whole((H, 4 * H)),
                whole((H, 4 * H)),
                whole((1, 4 * H)),
                whole((1, 4 * H)),
                whole((H, O)),
                whole((H, O)),
            ],
            out_specs=pl.BlockSpec((1, B, O),
                                   lambda d, s: (d, 0, t_of(d, s))),
            scratch_shapes=[
                pltpu.VMEM((I, 4 * H), jnp.bfloat16),
                pltpu.VMEM((H, 4 * H), f32),
                pltpu.VMEM((1, 4 * H), f32),
                pltpu.VMEM((H, O), f32),
                pltpu.VMEM((B, H), f32),
                pltpu.VMEM((B, H), f32),
            ],
        ),
        compiler_params=pltpu.CompilerParams(
            dimension_semantics=("arbitrary", "arbitrary")),
    )(x2, wi_f, wi_b, wh_f, wh_b, b_f, b_b, wl_f, wl_b)

    return (parts[0] + parts[1]).reshape(B, T, O) + b_lin


# speedup vs baseline: 2.0561x; 1.1665x over previous
import functools

import jax
import jax.numpy as jnp
from jax.experimental import pallas as pl
from jax.experimental.pallas import tpu as pltpu


def _bilstm_body(T, B, I, H, O,
                 x_ref,
                 wi_f_ref,
                 wi_b_ref,
                 wh_f_ref,
                 wh_b_ref,
                 b_f_ref,
                 b_b_ref,
                 wl_f_ref,
                 wl_b_ref,
                 bl_ref,
                 o_ref):
    f32 = jnp.float32
    bf16 = jnp.bfloat16

    col = jax.lax.broadcasted_iota(jnp.int32, (1, 4 * H), 1)
    gscale = jnp.where((col >= 2 * H) & (col < 3 * H), 1.0, 0.5).astype(f32)

    wi_f = wi_f_ref[...].astype(bf16)
    wi_b = wi_b_ref[...].astype(bf16)

    g_f = []
    g_b = []
    for t in range(T):
        xs = x_ref[:, t * I:(t + 1) * I].astype(bf16)
        g_f.append(jnp.dot(xs, wi_f, preferred_element_type=f32) + b_f_ref[...])
        g_b.append(jnp.dot(xs, wi_b, preferred_element_type=f32) + b_b_ref[...])

    def scan(gin, wh):
        h = jnp.zeros((B, H), f32)
        c = jnp.zeros((B, H), f32)
        hs = []
        for g in gin:
            gates = g + jnp.dot(h, wh, preferred_element_type=f32)
            th = jnp.tanh(gates * gscale)
            i_g = th[:, 0 * H:1 * H] * 0.5 + 0.5
            f_g = th[:, 1 * H:2 * H] * 0.5 + 0.5
            g_g = th[:, 2 * H:3 * H]
            o_g = th[:, 3 * H:4 * H] * 0.5 + 0.5
            c = f_g * c + i_g * g_g
            h = o_g * jnp.tanh(c)
            hs.append(h)
        return hs

    hs_f = scan(g_f, wh_f_ref[...])
    hs_b = scan(g_b[::-1], wh_b_ref[...])[::-1]

    wl_f = wl_f_ref[...]
    wl_b = wl_b_ref[...]
    bl = bl_ref[...]
    for t in range(T):
        o_ref[:, t * O:(t + 1) * O] = (
            jnp.dot(hs_f[t], wl_f, preferred_element_type=f32)
            + jnp.dot(hs_b[t], wl_b, preferred_element_type=f32) + bl)


@jax.jit
def kernel(x, wi_f, wh_f, b_f, wi_b, wh_b, b_b, wl_f, wl_b, b_lin):
    B, T, I = x.shape
    H = wh_f.shape[0]
    O = b_lin.shape[-1]
    f32 = jnp.float32

    x2 = x.reshape(B, T * I)

    def whole(shape):
        return pl.BlockSpec(shape, lambda i, _n=len(shape): (0,) * _n)

    out = pl.pallas_call(
        functools.partial(_bilstm_body, T, B, I, H, O),
        out_shape=jax.ShapeDtypeStruct((B, T * O), f32),
        grid_spec=pltpu.PrefetchScalarGridSpec(
            num_scalar_prefetch=0,
            grid=(1,),
            in_specs=[
                whole((B, T * I)),
                whole((I, 4 * H)),
                whole((I, 4 * H)),
                whole((H, 4 * H)),
                whole((H, 4 * H)),
                whole((1, 4 * H)),
                whole((1, 4 * H)),
                whole((H, O)),
                whole((H, O)),
                whole((1, O)),
            ],
            out_specs=whole((B, T * O)),
        ),
        compiler_params=pltpu.CompilerParams(
            dimension_semantics=("arbitrary",)),
    )(x2, wi_f, wi_b, wh_f, wh_b, b_f, b_b, wl_f, wl_b, b_lin)

    return out.reshape(B, T, O)
